# Initial kernel scaffold; baseline (speedup 1.0000x reference)
import jax
import jax.numpy as jnp
from jax.experimental import pallas as pl


def kernel(x, importance, w0, b0, w1, b1):
    raise NotImplementedError("write your pallas kernel here")


# speedup vs baseline: 3.9391x; 3.9391x over previous
import jax
import jax.numpy as jnp
from jax.experimental import pallas as pl
from jax.experimental.pallas import tpu as pltpu

KEEP_FRAC = 0.25
_ROWS = 256
_CHUNK = 256


def _mask_sumsq_kernel(imp_row_ref, imp_col_ref, x_ref,
                       mask_ref, sumsq_ref, acc_ref, mask_vmem):
    step = pl.program_id(0)
    nsteps = pl.num_programs(0)
    d = imp_row_ref.shape[1]
    k = max(1, int(d * KEEP_FRAC))

    @pl.when(step == 0)
    def _():
        iota_j = jax.lax.broadcasted_iota(jnp.int32, (1, d), 1)
        iota_chunk = jax.lax.broadcasted_iota(jnp.int32, (_CHUNK, 1), 0)
        imp_row = imp_row_ref[...]

        def body(c, rank):
            vi = imp_col_ref[pl.ds(c * _CHUNK, _CHUNK), :]
            ii = iota_chunk + c * _CHUNK
            beat = (vi > imp_row) | ((vi == imp_row) & (ii < iota_j))
            return rank + jnp.sum(beat.astype(jnp.int32), axis=0, keepdims=True)

        rank = jax.lax.fori_loop(0, d // _CHUNK, body,
                                 jnp.zeros((1, d), jnp.int32))
        m = (rank < k).astype(jnp.float32)
        mask_vmem[...] = m
        mask_ref[...] = m
        acc_ref[0] = 0.0

    xb = x_ref[...]
    unsel = 1.0 - mask_vmem[...]
    acc_ref[0] += jnp.sum(xb * xb * unsel)

    @pl.when(step == nsteps - 1)
    def _():
        sumsq_ref[0] = acc_ref[0]


def _apply_kernel(sumsq_ref, params_ref, mask_ref, x_ref, y_ref):
    loss = jnp.sqrt(sumsq_ref[0]) * 1e-6
    w0 = params_ref[0]
    b0 = params_ref[1]
    w1 = params_ref[2]
    b1 = params_ref[3]
    xb = x_ref[...]
    t = xb * w0 + b0
    t = t * jax.lax.logistic(t)
    t = t * w1 + b1
    t = t * jax.lax.logistic(t)
    m = mask_ref[...]
    y_ref[...] = xb + m * (t - xb) + loss


def kernel(x, importance, w0, b0, w1, b1):
    b_, t_, d_ = x.shape
    n = b_ * t_
    x2 = x.reshape(n, d_)
    imp_row = importance.reshape(1, d_)
    imp_col = importance.reshape(d_, 1)
    params = jnp.concatenate(
        [w0.reshape(-1), b0.reshape(-1), w1.reshape(-1), b1.reshape(-1)])
    nb = n // _ROWS

    mask, sumsq = pl.pallas_call(
        _mask_sumsq_kernel,
        grid=(nb,),
        in_specs=[
            pl.BlockSpec((1, d_), lambda b: (0, 0)),
            pl.BlockSpec((d_, 1), lambda b: (0, 0)),
            pl.BlockSpec((_ROWS, d_), lambda b: (b, 0)),
        ],
        out_specs=[
            pl.BlockSpec((1, d_), lambda b: (0, 0)),
            pl.BlockSpec(memory_space=pltpu.SMEM),
        ],
        out_shape=[
            jax.ShapeDtypeStruct((1, d_), jnp.float32),
            jax.ShapeDtypeStruct((1,), jnp.float32),
        ],
        scratch_shapes=[
            pltpu.SMEM((1,), jnp.float32),
            pltpu.VMEM((1, d_), jnp.float32),
        ],
    )(imp_row, imp_col, x2)

    y2 = pl.pallas_call(
        _apply_kernel,
        grid=(nb,),
        in_specs=[
            pl.BlockSpec(memory_space=pltpu.SMEM),
            pl.BlockSpec(memory_space=pltpu.SMEM),
            pl.BlockSpec((1, d_), lambda b: (0, 0)),
            pl.BlockSpec((_ROWS, d_), lambda b: (b, 0)),
        ],
        out_specs=pl.BlockSpec((_ROWS, d_), lambda b: (b, 0)),
        out_shape=jax.ShapeDtypeStruct((n, d_), jnp.float32),
    )(sumsq, params, mask, x2)

    return y2.reshape(b_, t_, d_)


# speedup vs baseline: 4.3344x; 1.1003x over previous
import jax
import jax.numpy as jnp
from jax.experimental import pallas as pl
from jax.experimental.pallas import tpu as pltpu

KEEP_FRAC = 0.25
_ROWS = 512
_CHUNK = 256


def _mask_sumsq_kernel(imp_row_ref, imp_col_ref, x_ref,
                       mask_ref, sumsq_ref, acc_ref, mask_vmem):
    step = pl.program_id(0)
    nsteps = pl.num_programs(0)
    d = imp_row_ref.shape[1]
    k = max(1, int(d * KEEP_FRAC))

    @pl.when(step == 0)
    def _():
        iota_j = jax.lax.broadcasted_iota(jnp.int32, (1, d), 1)
        iota_chunk = jax.lax.broadcasted_iota(jnp.int32, (_CHUNK, 1), 0)
        imp_row = imp_row_ref[...]

        def body(c, rank):
            vi = imp_col_ref[pl.ds(c * _CHUNK, _CHUNK), :]
            ii = iota_chunk + c * _CHUNK
            beat = (vi > imp_row) | ((vi == imp_row) & (ii < iota_j))
            return rank + jnp.sum(beat.astype(jnp.int32), axis=0, keepdims=True)

        rank = jax.lax.fori_loop(0, d // _CHUNK, body,
                                 jnp.zeros((1, d), jnp.int32))
        m = (rank < k).astype(jnp.float32)
        mask_vmem[...] = m
        mask_ref[...] = m
        acc_ref[0] = 0.0

    xb = x_ref[...]
    unsel = 1.0 - mask_vmem[...]
    acc_ref[0] += jnp.sum(xb * xb * unsel)

    @pl.when(step == nsteps - 1)
    def _():
        sumsq_ref[0] = acc_ref[0]


def _apply_kernel(sumsq_ref, params_ref, mask_ref, x_ref, y_ref):
    loss = jnp.sqrt(sumsq_ref[0]) * 1e-6
    w0 = params_ref[0]
    b0 = params_ref[1]
    w1 = params_ref[2]
    b1 = params_ref[3]
    xb = x_ref[...]
    t = xb * w0 + b0
    t = t * jax.lax.logistic(t)
    t = t * w1 + b1
    t = t * jax.lax.logistic(t)
    m = mask_ref[...]
    y_ref[...] = xb + m * (t - xb) + loss


def kernel(x, importance, w0, b0, w1, b1):
    b_, t_, d_ = x.shape
    n = b_ * t_
    x2 = x.reshape(n, d_)
    imp_row = importance.reshape(1, d_)
    imp_col = importance.reshape(d_, 1)
    params = jnp.concatenate(
        [w0.reshape(-1), b0.reshape(-1), w1.reshape(-1), b1.reshape(-1)])
    nb = n // _ROWS

    mask, sumsq = pl.pallas_call(
        _mask_sumsq_kernel,
        grid=(nb,),
        in_specs=[
            pl.BlockSpec((1, d_), lambda b: (0, 0)),
            pl.BlockSpec((d_, 1), lambda b: (0, 0)),
            pl.BlockSpec((_ROWS, d_), lambda b: (b, 0)),
        ],
        out_specs=[
            pl.BlockSpec((1, d_), lambda b: (0, 0)),
            pl.BlockSpec(memory_space=pltpu.SMEM),
        ],
        out_shape=[
            jax.ShapeDtypeStruct((1, d_), jnp.float32),
            jax.ShapeDtypeStruct((1,), jnp.float32),
        ],
        scratch_shapes=[
            pltpu.SMEM((1,), jnp.float32),
            pltpu.VMEM((1, d_), jnp.float32),
        ],
    )(imp_row, imp_col, x2)

    y2 = pl.pallas_call(
        _apply_kernel,
        grid=(nb,),
        in_specs=[
            pl.BlockSpec(memory_space=pltpu.SMEM),
            pl.BlockSpec(memory_space=pltpu.SMEM),
            pl.BlockSpec((1, d_), lambda b: (0, 0)),
            pl.BlockSpec((_ROWS, d_), lambda b: (b, 0)),
        ],
        out_specs=pl.BlockSpec((_ROWS, d_), lambda b: (b, 0)),
        out_shape=jax.ShapeDtypeStruct((n, d_), jnp.float32),
    )(sumsq, params, mask, x2)

    return y2.reshape(b_, t_, d_)


# speedup vs baseline: 4.4661x; 1.0304x over previous
import jax
import jax.numpy as jnp
from jax.experimental import pallas as pl
from jax.experimental.pallas import tpu as pltpu

KEEP_FRAC = 0.25
_ROWS = 1024
_CHUNK = 256


def _mask_sumsq_kernel(imp_row_ref, imp_col_ref, x_ref,
                       mask_ref, sumsq_ref, acc_ref, mask_vmem):
    step = pl.program_id(0)
    nsteps = pl.num_programs(0)
    d = imp_row_ref.shape[1]
    k = max(1, int(d * KEEP_FRAC))

    @pl.when(step == 0)
    def _():
        iota_j = jax.lax.broadcasted_iota(jnp.int32, (1, d), 1)
        iota_chunk = jax.lax.broadcasted_iota(jnp.int32, (_CHUNK, 1), 0)
        imp_row = imp_row_ref[...]

        def body(c, rank):
            vi = imp_col_ref[pl.ds(c * _CHUNK, _CHUNK), :]
            ii = iota_chunk + c * _CHUNK
            beat = (vi > imp_row) | ((vi == imp_row) & (ii < iota_j))
            return rank + jnp.sum(beat.astype(jnp.int32), axis=0, keepdims=True)

        rank = jax.lax.fori_loop(0, d // _CHUNK, body,
                                 jnp.zeros((1, d), jnp.int32))
        m = (rank < k).astype(jnp.float32)
        mask_vmem[...] = m
        mask_ref[...] = m
        acc_ref[0] = 0.0

    xb = x_ref[...]
    unsel = 1.0 - mask_vmem[...]
    acc_ref[0] += jnp.sum(xb * xb * unsel)

    @pl.when(step == nsteps - 1)
    def _():
        sumsq_ref[0] = acc_ref[0]


def _apply_kernel(sumsq_ref, params_ref, mask_ref, x_ref, y_ref):
    loss = jnp.sqrt(sumsq_ref[0]) * 1e-6
    w0 = params_ref[0]
    b0 = params_ref[1]
    w1 = params_ref[2]
    b1 = params_ref[3]
    xb = x_ref[...]
    t = xb * w0 + b0
    t = t * jax.lax.logistic(t)
    t = t * w1 + b1
    t = t * jax.lax.logistic(t)
    m = mask_ref[...]
    y_ref[...] = xb + m * (t - xb) + loss


def kernel(x, importance, w0, b0, w1, b1):
    b_, t_, d_ = x.shape
    n = b_ * t_
    x2 = x.reshape(n, d_)
    imp_row = importance.reshape(1, d_)
    imp_col = importance.reshape(d_, 1)
    params = jnp.concatenate(
        [w0.reshape(-1), b0.reshape(-1), w1.reshape(-1), b1.reshape(-1)])
    nb = n // _ROWS

    mask, sumsq = pl.pallas_call(
        _mask_sumsq_kernel,
        grid=(nb,),
        in_specs=[
            pl.BlockSpec((1, d_), lambda b: (0, 0)),
            pl.BlockSpec((d_, 1), lambda b: (0, 0)),
            pl.BlockSpec((_ROWS, d_), lambda b: (b, 0)),
        ],
        out_specs=[
            pl.BlockSpec((1, d_), lambda b: (0, 0)),
            pl.BlockSpec(memory_space=pltpu.SMEM),
        ],
        out_shape=[
            jax.ShapeDtypeStruct((1, d_), jnp.float32),
            jax.ShapeDtypeStruct((1,), jnp.float32),
        ],
        scratch_shapes=[
            pltpu.SMEM((1,), jnp.float32),
            pltpu.VMEM((1, d_), jnp.float32),
        ],
    )(imp_row, imp_col, x2)

    y2 = pl.pallas_call(
        _apply_kernel,
        grid=(nb,),
        in_specs=[
            pl.BlockSpec(memory_space=pltpu.SMEM),
            pl.BlockSpec(memory_space=pltpu.SMEM),
            pl.BlockSpec((1, d_), lambda b: (0, 0)),
            pl.BlockSpec((_ROWS, d_), lambda b: (b, 0)),
        ],
        out_specs=pl.BlockSpec((_ROWS, d_), lambda b: (b, 0)),
        out_shape=jax.ShapeDtypeStruct((n, d_), jnp.float32),
    )(sumsq, params, mask, x2)

    return y2.reshape(b_, t_, d_)
